# baseline (speedup 1.0000x reference)
import functools

import jax
import jax.numpy as jnp
from jax import lax
from jax.experimental import pallas as pl
from jax.experimental.pallas import tpu as pltpu
from jax.experimental.pallas import tpu_sc as plsc

NC = 2
NS = 16
L = 16
NW = NC * NS

CH = 128
ZR = 128
R = 512


def _deg_body(npad, ncht, ei_hbm, z_hbm, deg_hbm, idx, ones_v, isem,
              deg_sp):
    pers = npad // NS
    c = lax.axis_index("c")
    s = lax.axis_index("s")
    w = c * NS + s
    nfull = ncht // NW
    nrem = ncht - nfull * NW
    nch_w = nfull + jnp.where(w < nrem, 1, 0)

    def ibody(i, carry):
        ones_v[pl.ds(i * L, L)] = jnp.ones((L,), jnp.float32)
        return carry

    lax.fori_loop(0, CH // L, ibody, 0)
    pltpu.sync_copy(z_hbm.at[pl.ds(s * pers, pers)],
                    deg_sp.at[pl.ds(s * pers, pers)])
    plsc.subcore_barrier()

    for q in range(4):
        pltpu.async_copy(ei_hbm.at[:, pl.ds((q * NW + w) * CH, CH)],
                         idx.at[q], isem)

    def jbody(j, carry):
        ib = lax.rem(j, 4)
        pltpu.make_async_copy(ei_hbm.at[:, pl.ds((j * NW + w) * CH, CH)],
                              idx.at[ib], isem).wait()
        pltpu.sync_copy(ones_v, deg_sp.at[idx.at[ib, 1]], add=True)

        @pl.when(j + 4 < nch_w)
        def _():
            pltpu.async_copy(
                ei_hbm.at[:, pl.ds(((j + 4) * NW + w) * CH, CH)],
                idx.at[ib], isem)

        return carry

    lax.fori_loop(0, nch_w, jbody, 0)
    plsc.subcore_barrier()
    pltpu.sync_copy(deg_sp.at[pl.ds(s * pers, pers)],
                    deg_hbm.at[c, pl.ds(s * pers, pers)])


def _scatter_body(npad, ncht, d, x2_hbm, ei_hbm, z_hbm, agg_hbm,
                  idx, rows, isem, gsem, ssem, agg_sp):
    pers = npad // NS
    c = lax.axis_index("c")
    s = lax.axis_index("s")
    w = c * NS + s
    nfull = ncht // NW
    nrem = ncht - nfull * NW
    nch_w = nfull + jnp.where(w < nrem, 1, 0)

    for q in range(pers // ZR):
        pltpu.sync_copy(z_hbm, agg_sp.at[pl.ds(s * pers + q * ZR, ZR)])
    plsc.subcore_barrier()

    pltpu.async_copy(ei_hbm.at[:, pl.ds(w * CH, CH)], idx.at[0],
                     isem).wait()
    pltpu.async_copy(x2_hbm.at[idx.at[0, 0]], rows.at[0], gsem)
    pltpu.async_copy(ei_hbm.at[:, pl.ds((NW + w) * CH, CH)], idx.at[1],
                     isem)

    def jbody(j, carry):
        b = lax.rem(j, 2)
        nb = 1 - b
        ib = lax.rem(j, 4)
        pltpu.make_async_copy(x2_hbm.at[idx.at[ib, 0]], rows.at[b],
                              gsem).wait()

        @pl.when(j >= 1)
        def _():
            pltpu.make_async_copy(rows.at[nb],
                                  agg_sp.at[idx.at[lax.rem(j + 3, 4), 1]],
                                  ssem).wait()

        pltpu.async_copy(rows.at[b], agg_sp.at[idx.at[ib, 1]], ssem,
                         add=True)

        @pl.when(j + 1 < nch_w)
        def _():
            nib = lax.rem(j + 1, 4)
            pltpu.make_async_copy(
                ei_hbm.at[:, pl.ds(((j + 1) * NW + w) * CH, CH)],
                idx.at[nib], isem).wait()
            pltpu.async_copy(x2_hbm.at[idx.at[nib, 0]], rows.at[nb], gsem)

        @pl.when(j + 2 < nch_w)
        def _():
            pltpu.async_copy(
                ei_hbm.at[:, pl.ds(((j + 2) * NW + w) * CH, CH)],
                idx.at[lax.rem(j + 2, 4)], isem)

        return carry

    lax.fori_loop(0, nch_w, jbody, 0)
    pltpu.make_async_copy(rows.at[lax.rem(nch_w - 1, 2)],
                          agg_sp.at[idx.at[lax.rem(nch_w - 1, 4), 1]],
                          ssem).wait()
    plsc.subcore_barrier()
    pltpu.sync_copy(agg_sp.at[pl.ds(s * pers, pers)],
                    agg_hbm.at[c, pl.ds(s * pers, pers)])


def _scale_body(x_ref, deg_ref, o_ref):
    dg = deg_ref[...]
    dinv = lax.rsqrt(dg[0:1, :] + dg[1:2, :] + 1.0)
    o_ref[...] = x_ref[...] * jnp.transpose(dinv)


def _final_body(agg_ref, x2_ref, deg_ref, w_ref, b_ref, o_ref):
    dg = deg_ref[...]
    dinv = lax.rsqrt(dg[0:1, :] + dg[1:2, :] + 1.0)
    pre = (agg_ref[0] + agg_ref[1] + x2_ref[...]) * jnp.transpose(dinv)
    o_ref[...] = (jnp.dot(pre, w_ref[...], preferred_element_type=jnp.float32)
                  + b_ref[...])


@jax.jit
def kernel(x, edge_index, W, b):
    n, d = x.shape
    e = edge_index.shape[1]
    assert e % CH == 0
    ncht = e // CH
    npad = ((n + NS * L - 1) // (NS * L)) * (NS * L)

    z1 = jnp.zeros((npad,), jnp.float32)
    z2 = jnp.zeros((ZR, d), jnp.float32)

    mesh = plsc.VectorSubcoreMesh(core_axis_name="c", subcore_axis_name="s")

    deg = pl.kernel(
        functools.partial(_deg_body, npad, ncht),
        out_type=jax.ShapeDtypeStruct((NC, npad), jnp.float32),
        mesh=mesh,
        scratch_types=[
            pltpu.VMEM((4, 2, CH), jnp.int32),
            pltpu.VMEM((CH,), jnp.float32),
            pltpu.SemaphoreType.DMA,
            pltpu.VMEM_SHARED((npad,), jnp.float32),
        ],
    )(edge_index, z1)

    nblk = npad // R
    x2 = pl.pallas_call(
        _scale_body,
        grid=(nblk,),
        in_specs=[
            pl.BlockSpec((R, d), lambda i: (i, 0)),
            pl.BlockSpec((NC, R), lambda i: (0, i)),
        ],
        out_specs=pl.BlockSpec((R, d), lambda i: (i, 0)),
        out_shape=jax.ShapeDtypeStruct((n, d), jnp.float32),
    )(x, deg)

    agg = pl.kernel(
        functools.partial(_scatter_body, npad, ncht, d),
        out_type=jax.ShapeDtypeStruct((NC, npad, d), jnp.float32),
        mesh=mesh,
        scratch_types=[
            pltpu.VMEM((4, 2, CH), jnp.int32),
            pltpu.VMEM((2, CH, d), jnp.float32),
            pltpu.SemaphoreType.DMA,
            pltpu.SemaphoreType.DMA,
            pltpu.SemaphoreType.DMA,
            pltpu.VMEM_SHARED((npad, d), jnp.float32),
        ],
    )(x2, edge_index, z2)

    out = pl.pallas_call(
        _final_body,
        grid=(nblk,),
        in_specs=[
            pl.BlockSpec((NC, R, d), lambda i: (0, i, 0)),
            pl.BlockSpec((R, d), lambda i: (i, 0)),
            pl.BlockSpec((NC, R), lambda i: (0, i)),
            pl.BlockSpec((d, d), lambda i: (0, 0)),
            pl.BlockSpec((1, d), lambda i: (0, 0)),
        ],
        out_specs=pl.BlockSpec((R, d), lambda i: (i, 0)),
        out_shape=jax.ShapeDtypeStruct((n, d), jnp.float32),
    )(agg, x2, deg, W, b.reshape(1, d))

    return out


# speedup vs baseline: 1.0307x; 1.0307x over previous
import functools

import jax
import jax.numpy as jnp
from jax import lax
from jax.experimental import pallas as pl
from jax.experimental.pallas import tpu as pltpu
from jax.experimental.pallas import tpu_sc as plsc

NC = 2
NS = 16
L = 16
NW = NC * NS

CH = 128
R = 512


def _deg_body(npad, ncht, ei_hbm, z_hbm, deg_hbm, idx, ones_v, isem,
              deg_sp):
    pers = npad // NS
    c = lax.axis_index("c")
    s = lax.axis_index("s")
    w = c * NS + s
    nfull = ncht // NW
    nrem = ncht - nfull * NW
    nch_w = nfull + jnp.where(w < nrem, 1, 0)

    def ibody(i, carry):
        ones_v[pl.ds(i * L, L)] = jnp.ones((L,), jnp.float32)
        return carry

    lax.fori_loop(0, CH // L, ibody, 0)
    pltpu.sync_copy(z_hbm.at[pl.ds(s * pers, pers)],
                    deg_sp.at[pl.ds(s * pers, pers)])
    plsc.subcore_barrier()

    for q in range(4):
        pltpu.async_copy(ei_hbm.at[:, pl.ds((q * NW + w) * CH, CH)],
                         idx.at[q], isem)

    def jbody(j, carry):
        ib = lax.rem(j, 4)
        pltpu.make_async_copy(ei_hbm.at[:, pl.ds((j * NW + w) * CH, CH)],
                              idx.at[ib], isem).wait()
        pltpu.sync_copy(ones_v, deg_sp.at[idx.at[ib, 1]], add=True)

        @pl.when(j + 4 < nch_w)
        def _():
            pltpu.async_copy(
                ei_hbm.at[:, pl.ds(((j + 4) * NW + w) * CH, CH)],
                idx.at[ib], isem)

        return carry

    lax.fori_loop(0, nch_w, jbody, 0)
    plsc.subcore_barrier()
    pltpu.sync_copy(deg_sp.at[pl.ds(s * pers, pers)],
                    deg_hbm.at[c, pl.ds(s * pers, pers)])


def _scatter_body(npad, ncht, d, x2_hbm, ei_hbm, z_hbm, agg_hbm,
                  idx, rows, isem, gsem, ssem, agg_sp):
    pers = npad // NS
    c = lax.axis_index("c")
    s = lax.axis_index("s")
    w = c * NS + s
    nfull = ncht // NW
    nrem = ncht - nfull * NW
    nch_w = nfull + jnp.where(w < nrem, 1, 0)

    pltpu.sync_copy(z_hbm.at[pl.ds(s * pers, pers)],
                    agg_sp.at[pl.ds(s * pers, pers)])
    plsc.subcore_barrier()

    pltpu.async_copy(ei_hbm.at[:, pl.ds(w * CH, CH)], idx.at[0],
                     isem).wait()
    pltpu.async_copy(x2_hbm.at[idx.at[0, 0]], rows.at[0], gsem)
    pltpu.async_copy(ei_hbm.at[:, pl.ds((NW + w) * CH, CH)], idx.at[1],
                     isem)

    def jbody(j, carry):
        b = lax.rem(j, 2)
        nb = 1 - b
        ib = lax.rem(j, 4)
        pltpu.make_async_copy(x2_hbm.at[idx.at[ib, 0]], rows.at[b],
                              gsem).wait()

        @pl.when(j >= 1)
        def _():
            pltpu.make_async_copy(rows.at[nb],
                                  agg_sp.at[idx.at[lax.rem(j + 3, 4), 1]],
                                  ssem).wait()

        pltpu.async_copy(rows.at[b], agg_sp.at[idx.at[ib, 1]], ssem,
                         add=True)

        @pl.when(j + 1 < nch_w)
        def _():
            nib = lax.rem(j + 1, 4)
            pltpu.make_async_copy(
                ei_hbm.at[:, pl.ds(((j + 1) * NW + w) * CH, CH)],
                idx.at[nib], isem).wait()
            pltpu.async_copy(x2_hbm.at[idx.at[nib, 0]], rows.at[nb], gsem)

        @pl.when(j + 2 < nch_w)
        def _():
            pltpu.async_copy(
                ei_hbm.at[:, pl.ds(((j + 2) * NW + w) * CH, CH)],
                idx.at[lax.rem(j + 2, 4)], isem)

        return carry

    lax.fori_loop(0, nch_w, jbody, 0)
    pltpu.make_async_copy(rows.at[lax.rem(nch_w - 1, 2)],
                          agg_sp.at[idx.at[lax.rem(nch_w - 1, 4), 1]],
                          ssem).wait()
    plsc.subcore_barrier()
    pltpu.sync_copy(agg_sp.at[pl.ds(s * pers, pers)],
                    agg_hbm.at[c, pl.ds(s * pers, pers)])


def _scale_body(x_ref, deg_ref, o_ref):
    dg = deg_ref[...]
    dinv = lax.rsqrt(dg[0:1, :] + dg[1:2, :] + 1.0)
    o_ref[...] = x_ref[...] * jnp.transpose(dinv)


def _final_body(agg_ref, x2_ref, deg_ref, w_ref, b_ref, o_ref):
    dg = deg_ref[...]
    dinv = lax.rsqrt(dg[0:1, :] + dg[1:2, :] + 1.0)
    pre = (agg_ref[0] + agg_ref[1] + x2_ref[...]) * jnp.transpose(dinv)
    o_ref[...] = (jnp.dot(pre, w_ref[...], preferred_element_type=jnp.float32)
                  + b_ref[...])


@jax.jit
def kernel(x, edge_index, W, b):
    n, d = x.shape
    e = edge_index.shape[1]
    assert e % CH == 0
    ncht = e // CH
    npad = ((n + NS * L - 1) // (NS * L)) * (NS * L)

    z1 = jnp.zeros((npad,), jnp.float32)
    z2 = jnp.zeros((npad, d), jnp.float32)

    mesh = plsc.VectorSubcoreMesh(core_axis_name="c", subcore_axis_name="s")

    deg = pl.kernel(
        functools.partial(_deg_body, npad, ncht),
        out_type=jax.ShapeDtypeStruct((NC, npad), jnp.float32),
        mesh=mesh,
        scratch_types=[
            pltpu.VMEM((4, 2, CH), jnp.int32),
            pltpu.VMEM((CH,), jnp.float32),
            pltpu.SemaphoreType.DMA,
            pltpu.VMEM_SHARED((npad,), jnp.float32),
        ],
    )(edge_index, z1)

    nblk = npad // R
    x2 = pl.pallas_call(
        _scale_body,
        grid=(nblk,),
        in_specs=[
            pl.BlockSpec((R, d), lambda i: (i, 0)),
            pl.BlockSpec((NC, R), lambda i: (0, i)),
        ],
        out_specs=pl.BlockSpec((R, d), lambda i: (i, 0)),
        out_shape=jax.ShapeDtypeStruct((n, d), jnp.float32),
    )(x, deg)

    agg = pl.kernel(
        functools.partial(_scatter_body, npad, ncht, d),
        out_type=jax.ShapeDtypeStruct((NC, npad, d), jnp.float32),
        mesh=mesh,
        scratch_types=[
            pltpu.VMEM((4, 2, CH), jnp.int32),
            pltpu.VMEM((2, CH, d), jnp.float32),
            pltpu.SemaphoreType.DMA,
            pltpu.SemaphoreType.DMA,
            pltpu.SemaphoreType.DMA,
            pltpu.VMEM_SHARED((npad, d), jnp.float32),
        ],
    )(x2, edge_index, z2)

    out = pl.pallas_call(
        _final_body,
        grid=(nblk,),
        in_specs=[
            pl.BlockSpec((NC, R, d), lambda i: (0, i, 0)),
            pl.BlockSpec((R, d), lambda i: (i, 0)),
            pl.BlockSpec((NC, R), lambda i: (0, i)),
            pl.BlockSpec((d, d), lambda i: (0, 0)),
            pl.BlockSpec((1, d), lambda i: (0, 0)),
        ],
        out_specs=pl.BlockSpec((R, d), lambda i: (i, 0)),
        out_shape=jax.ShapeDtypeStruct((n, d), jnp.float32),
    )(agg, x2, deg, W, b.reshape(1, d))

    return out
